# Initial kernel scaffold; baseline (speedup 1.0000x reference)
import jax
import jax.numpy as jnp
from jax.experimental import pallas as pl


def kernel(x, adj, W1, W2):
    raise NotImplementedError("write your pallas kernel here")


# speedup vs baseline: 1.2757x; 1.2757x over previous
import functools

import jax
import jax.numpy as jnp
from jax.experimental import pallas as pl
from jax.experimental.pallas import tpu as pltpu


def _pick_bm(n):
    for c in (400, 200, 80, 40, 16, 8):
        if n % c == 0:
            return c
    return n


def _proj_kernel(nh, x_ref, c1_ref, s1_ref, p1_ref):
    z = jnp.dot(x_ref[...], c1_ref[...], preferred_element_type=jnp.float32)
    s1_ref[...] = z[:, :nh]
    p1_ref[...] = z[:, nh:]


def _l1_kernel(ne, adj_ref, p1_ref, s1_ref, c2_ref, s2_ref, p2_ref, deg_ref):
    adj = adj_ref[...]
    agg = jnp.dot(adj.astype(jnp.bfloat16), p1_ref[...],
                  preferred_element_type=jnp.float32)
    deg = jnp.sum(adj, axis=1, keepdims=True) + 1.0
    h = jnp.maximum(s1_ref[...] + agg / deg, 0.0)
    z = jnp.dot(h, c2_ref[...], preferred_element_type=jnp.float32)
    s2_ref[...] = z[:, :ne]
    p2_ref[...] = z[:, ne:]
    deg_ref[...] = deg


def _l2_kernel(adj_ref, p2_ref, s2_ref, deg_ref, o_ref):
    agg = jnp.dot(adj_ref[...].astype(jnp.bfloat16), p2_ref[...],
                  preferred_element_type=jnp.float32)
    o_ref[...] = jnp.maximum(s2_ref[...] + agg / deg_ref[...], 0.0)


def kernel(x, adj, W1, W2):
    n, nf = x.shape
    nh = W1.shape[0]
    ne = W2.shape[0]
    bm = _pick_bm(n)

    c1 = jnp.concatenate([W1[:, :nf].T, W1[:, nf:].T], axis=1)
    c2 = jnp.concatenate([W2[:, :nh].T, W2[:, nh:].T], axis=1)

    bp = _pick_bm(n)
    s1, p1 = pl.pallas_call(
        functools.partial(_proj_kernel, nh),
        grid=(n // bp,),
        in_specs=[
            pl.BlockSpec((bp, nf), lambda i: (i, 0)),
            pl.BlockSpec((nf, 2 * nh), lambda i: (0, 0)),
        ],
        out_specs=[
            pl.BlockSpec((bp, nh), lambda i: (i, 0)),
            pl.BlockSpec((bp, nh), lambda i: (i, 0)),
        ],
        out_shape=[
            jax.ShapeDtypeStruct((n, nh), jnp.float32),
            jax.ShapeDtypeStruct((n, nh), jnp.float32),
        ],
    )(x, c1)

    p1h = p1.astype(jnp.bfloat16)
    s2, p2, deg = pl.pallas_call(
        functools.partial(_l1_kernel, ne),
        grid=(n // bm,),
        in_specs=[
            pl.BlockSpec((bm, n), lambda i: (i, 0)),
            pl.BlockSpec((n, nh), lambda i: (0, 0)),
            pl.BlockSpec((bm, nh), lambda i: (i, 0)),
            pl.BlockSpec((nh, 2 * ne), lambda i: (0, 0)),
        ],
        out_specs=[
            pl.BlockSpec((bm, ne), lambda i: (i, 0)),
            pl.BlockSpec((bm, ne), lambda i: (i, 0)),
            pl.BlockSpec((bm, 1), lambda i: (i, 0)),
        ],
        out_shape=[
            jax.ShapeDtypeStruct((n, ne), jnp.float32),
            jax.ShapeDtypeStruct((n, ne), jnp.float32),
            jax.ShapeDtypeStruct((n, 1), jnp.float32),
        ],
        compiler_params=pltpu.CompilerParams(
            dimension_semantics=("arbitrary",)),
    )(adj, p1h, s1, c2)

    p2h = p2.astype(jnp.bfloat16)
    out = pl.pallas_call(
        _l2_kernel,
        grid=(n // bm,),
        in_specs=[
            pl.BlockSpec((bm, n), lambda i: (i, 0)),
            pl.BlockSpec((n, ne), lambda i: (0, 0)),
            pl.BlockSpec((bm, ne), lambda i: (i, 0)),
            pl.BlockSpec((bm, 1), lambda i: (i, 0)),
        ],
        out_specs=pl.BlockSpec((bm, ne), lambda i: (i, 0)),
        out_shape=jax.ShapeDtypeStruct((n, ne), jnp.float32),
        compiler_params=pltpu.CompilerParams(
            dimension_semantics=("arbitrary",)),
    )(adj, p2h, s2, deg)
    return out


# speedup vs baseline: 1.4611x; 1.1453x over previous
import functools

import jax
import jax.numpy as jnp
from jax.experimental import pallas as pl
from jax.experimental.pallas import tpu as pltpu


def _pick_bm(n):
    for c in (400, 200, 80, 40, 16, 8):
        if n % c == 0:
            return c
    return n


def _l1_kernel(ne, bm, adj_ref, x_ref, c1_ref, c2_ref,
               s2_ref, p2_ref, deg_ref, p1_ref):
    i = pl.program_id(0)
    nf = x_ref.shape[1]

    @pl.when(i == 0)
    def _():
        p1_ref[...] = jnp.dot(
            x_ref[...], c1_ref[:, nf:], preferred_element_type=jnp.float32
        ).astype(jnp.bfloat16)

    adj = adj_ref[...]
    agg = jnp.dot(adj.astype(jnp.bfloat16), p1_ref[...],
                  preferred_element_type=jnp.float32)
    deg = jnp.sum(adj, axis=1, keepdims=True) + 1.0
    x_blk = x_ref[pl.ds(i * bm, bm), :]
    s1 = jnp.dot(x_blk, c1_ref[:, :nf], preferred_element_type=jnp.float32)
    h = jnp.maximum(s1 + agg / deg, 0.0)
    z = jnp.dot(h, c2_ref[...], preferred_element_type=jnp.float32)
    s2_ref[...] = z[:, :ne]
    p2_ref[...] = z[:, ne:].astype(jnp.bfloat16)
    deg_ref[...] = deg


def _l2_kernel(adj_ref, p2_ref, s2_ref, deg_ref, o_ref):
    agg = jnp.dot(adj_ref[...].astype(jnp.bfloat16), p2_ref[...],
                  preferred_element_type=jnp.float32)
    o_ref[...] = jnp.maximum(s2_ref[...] + agg / deg_ref[...], 0.0)


def kernel(x, adj, W1, W2):
    n, nf = x.shape
    nh = W1.shape[0]
    ne = W2.shape[0]
    bm = _pick_bm(n)

    c1 = jnp.concatenate([W1[:, :nf].T, W1[:, nf:].T], axis=1)
    c2 = jnp.concatenate([W2[:, :nh].T, W2[:, nh:].T], axis=1)

    s2, p2, deg = pl.pallas_call(
        functools.partial(_l1_kernel, ne, bm),
        grid=(n // bm,),
        in_specs=[
            pl.BlockSpec((bm, n), lambda i: (i, 0)),
            pl.BlockSpec((n, nf), lambda i: (0, 0)),
            pl.BlockSpec((nf, 2 * nh), lambda i: (0, 0)),
            pl.BlockSpec((nh, 2 * ne), lambda i: (0, 0)),
        ],
        out_specs=[
            pl.BlockSpec((bm, ne), lambda i: (i, 0)),
            pl.BlockSpec((bm, ne), lambda i: (i, 0)),
            pl.BlockSpec((bm, 1), lambda i: (i, 0)),
        ],
        out_shape=[
            jax.ShapeDtypeStruct((n, ne), jnp.float32),
            jax.ShapeDtypeStruct((n, ne), jnp.bfloat16),
            jax.ShapeDtypeStruct((n, 1), jnp.float32),
        ],
        scratch_shapes=[pltpu.VMEM((n, nh), jnp.bfloat16)],
        compiler_params=pltpu.CompilerParams(
            dimension_semantics=("arbitrary",)),
    )(adj, x, c1, c2)

    out = pl.pallas_call(
        _l2_kernel,
        grid=(n // bm,),
        in_specs=[
            pl.BlockSpec((bm, n), lambda i: (i, 0)),
            pl.BlockSpec((n, ne), lambda i: (0, 0)),
            pl.BlockSpec((bm, ne), lambda i: (i, 0)),
            pl.BlockSpec((bm, 1), lambda i: (i, 0)),
        ],
        out_specs=pl.BlockSpec((bm, ne), lambda i: (i, 0)),
        out_shape=jax.ShapeDtypeStruct((n, ne), jnp.float32),
        compiler_params=pltpu.CompilerParams(
            dimension_semantics=("arbitrary",)),
    )(adj, p2, s2, deg)
    return out
